# Initial kernel scaffold; baseline (speedup 1.0000x reference)
import jax
import jax.numpy as jnp
from jax.experimental import pallas as pl


def kernel(text_features, audio_features, vision_features, text_labels, audio_labels, vision_labels, fused_representations, learnable_class_centers):
    raise NotImplementedError("write your pallas kernel here")


# speedup vs baseline: 12.8863x; 12.8863x over previous
import functools

import jax
import jax.numpy as jnp
from jax.experimental import pallas as pl

B = 512
D = 512
C = 3
DELTA = 1.5


def _fused_kernel(tf_ref, af_ref, vf_ref, tl_ref, al_ref, vl_ref,
                  fr_ref, lc_ref, adj_ref, nf_ref):
    wacc = jnp.zeros((B, B), dtype=jnp.float32)
    centers_sum = jnp.zeros((C, D), dtype=jnp.float32)

    for f_ref, l_ref in ((tf_ref, tl_ref), (af_ref, al_ref), (vf_ref, vl_ref)):
        feats = f_ref[:]
        labels = l_ref[:]

        pred = jnp.argmax(labels, axis=1)
        onehot = (pred[:, None] == jnp.arange(C)[None, :]).astype(jnp.float32)

        counts = jnp.sum(onehot, axis=0)
        centers = jnp.dot(onehot.T, feats,
                          preferred_element_type=jnp.float32)
        centers = centers / jnp.maximum(counts, 1.0)[:, None]
        centers_sum = centers_sum + centers

        logp = jnp.log(labels)
        h = jnp.sum(labels * logp, axis=1)
        cross1 = jnp.dot(labels, logp.T,
                         preferred_element_type=jnp.float32)
        cross2 = jnp.dot(logp, labels.T,
                         preferred_element_type=jnp.float32)
        skl = 0.5 * (h[:, None] + h[None, :] - cross1 - cross2)

        f2 = jnp.sum(feats * feats, axis=1)
        c2 = jnp.sum(centers * centers, axis=1)
        G = jnp.dot(feats, centers.T,
                    preferred_element_type=jnp.float32)
        g = jnp.exp(-0.5 * (f2[:, None] + c2[None, :] - 2.0 * G))

        prod = jnp.zeros((B, B), dtype=jnp.float32)
        dij = jnp.zeros((B, B), dtype=jnp.float32)
        dji = jnp.zeros((B, B), dtype=jnp.float32)
        for k in range(C):
            dij = dij + g[:, k][:, None] * onehot[:, k][None, :]
            dji = dji + onehot[:, k][:, None] * g[:, k][None, :]
        prod = dij * dji

        wacc = wacc + jnp.where(skl < DELTA,
                                (1.0 - skl / DELTA) * prod, 0.0)

    w = wacc * (1.0 / 3.0)
    ri = jax.lax.broadcasted_iota(jnp.int32, (B, B), 0)
    ci = jax.lax.broadcasted_iota(jnp.int32, (B, B), 1)
    w = jnp.where(ri == ci, 0.0, w)

    fused = fr_ref[:]
    lcent = lc_ref[:]
    logits = jnp.dot(fused, lcent.T,
                     preferred_element_type=jnp.float32)
    pred_f = jnp.argmax(logits, axis=1)
    onehot_f = (pred_f[:, None] == jnp.arange(C)[None, :]).astype(jnp.float32)

    avg_c = centers_sum * (1.0 / 3.0)
    fu2 = jnp.sum(fused * fused, axis=1)
    a2 = jnp.sum(avg_c * avg_c, axis=1)
    Gf = jnp.dot(fused, avg_c.T,
                 preferred_element_type=jnp.float32)
    d2f = fu2 + jnp.sum(onehot_f * (a2[None, :] - 2.0 * Gf), axis=1)
    wf = jnp.exp(-0.5 * d2f)

    R = wf[:, None] * onehot_f

    adj_ref[0:B, 0:B] = w
    adj_ref[0:B, B:B + C] = R
    adj_ref[B:B + C, 0:B] = R.T
    adj_ref[B:B + C, B:B + C] = jnp.zeros((C, C), dtype=jnp.float32)

    nf_ref[0:B, :] = fused
    nf_ref[B:B + C, :] = lcent


@functools.partial(jax.jit)
def kernel(text_features, audio_features, vision_features, text_labels,
           audio_labels, vision_labels, fused_representations,
           learnable_class_centers):
    n = B + C
    adj, node_features = pl.pallas_call(
        _fused_kernel,
        out_shape=(
            jax.ShapeDtypeStruct((n, n), jnp.float32),
            jax.ShapeDtypeStruct((n, D), jnp.float32),
        ),
    )(text_features, audio_features, vision_features,
      text_labels, audio_labels, vision_labels,
      fused_representations, learnable_class_centers)
    return adj, node_features


# speedup vs baseline: 14.7615x; 1.1455x over previous
import functools

import jax
import jax.numpy as jnp
from jax.experimental import pallas as pl

B = 512
D = 512
C = 3
DELTA = 1.5


def _fused_kernel(tf_ref, af_ref, vf_ref, tl_ref, al_ref, vl_ref,
                  fr_ref, lc_ref, adj_ref, nf_ref):
    wacc = jnp.zeros((B, B), dtype=jnp.float32)
    centers_sum = jnp.zeros((C, D), dtype=jnp.float32)
    s = 0.5 / (3.0 * DELTA)

    for f_ref, l_ref in ((tf_ref, tl_ref), (af_ref, al_ref), (vf_ref, vl_ref)):
        feats = f_ref[:]
        labels = l_ref[:]

        pred = jnp.argmax(labels, axis=1)
        onehot = (pred[:, None] == jnp.arange(C)[None, :]).astype(jnp.float32)

        counts = jnp.sum(onehot, axis=0)
        centers = jnp.dot(onehot.T, feats,
                          preferred_element_type=jnp.float32)
        centers = centers / jnp.maximum(counts, 1.0)[:, None]
        centers_sum = centers_sum + centers

        logp = jnp.log(labels)
        h = jnp.sum(labels * logp, axis=1, keepdims=True)
        ones = jnp.ones((B, 1), dtype=jnp.float32)
        U = jnp.concatenate([labels, logp, h, ones], axis=1)
        V = jnp.concatenate([s * logp, s * labels, -s * ones,
                             1.0 / 3.0 - s * h], axis=1)
        T = jnp.dot(U, V.T, preferred_element_type=jnp.float32)

        f2 = jnp.sum(feats * feats, axis=1)
        c2 = jnp.sum(centers * centers, axis=1)
        G = jnp.dot(feats, centers.T,
                    preferred_element_type=jnp.float32)
        g = jnp.exp(-0.5 * (f2[:, None] + c2[None, :] - 2.0 * G))

        dij = jnp.dot(g, onehot.T, preferred_element_type=jnp.float32)
        dji = jnp.dot(onehot, g.T, preferred_element_type=jnp.float32)

        wacc = wacc + jnp.maximum(T, 0.0) * dij * dji

    ri = jax.lax.broadcasted_iota(jnp.int32, (B, B), 0)
    ci = jax.lax.broadcasted_iota(jnp.int32, (B, B), 1)
    w = jnp.where(ri == ci, 0.0, wacc)

    fused = fr_ref[:]
    lcent = lc_ref[:]
    logits = jnp.dot(fused, lcent.T,
                     preferred_element_type=jnp.float32)
    pred_f = jnp.argmax(logits, axis=1)
    onehot_f = (pred_f[:, None] == jnp.arange(C)[None, :]).astype(jnp.float32)

    avg_c = centers_sum * (1.0 / 3.0)
    fu2 = jnp.sum(fused * fused, axis=1)
    a2 = jnp.sum(avg_c * avg_c, axis=1)
    Gf = jnp.dot(fused, avg_c.T,
                 preferred_element_type=jnp.float32)
    d2f = fu2 + jnp.sum(onehot_f * (a2[None, :] - 2.0 * Gf), axis=1)
    wf = jnp.exp(-0.5 * d2f)

    R = wf[:, None] * onehot_f

    adj_ref[0:B, 0:B] = w
    adj_ref[0:B, B:B + C] = R
    adj_ref[B:B + C, 0:B] = R.T
    adj_ref[B:B + C, B:B + C] = jnp.zeros((C, C), dtype=jnp.float32)

    nf_ref[0:B, :] = fused
    nf_ref[B:B + C, :] = lcent


@functools.partial(jax.jit)
def kernel(text_features, audio_features, vision_features, text_labels,
           audio_labels, vision_labels, fused_representations,
           learnable_class_centers):
    n = B + C
    adj, node_features = pl.pallas_call(
        _fused_kernel,
        out_shape=(
            jax.ShapeDtypeStruct((n, n), jnp.float32),
            jax.ShapeDtypeStruct((n, D), jnp.float32),
        ),
    )(text_features, audio_features, vision_features,
      text_labels, audio_labels, vision_labels,
      fused_representations, learnable_class_centers)
    return adj, node_features
